# baseline (speedup 1.0000x reference)
import functools

import jax
import jax.numpy as jnp
from jax import lax
from jax.experimental import pallas as pl
from jax.experimental.pallas import tpu as pltpu
from jax.experimental.pallas import tpu_sc as plsc

_B = 16384
_DIM = 128
_NC = 2
_NS = 16
_NW = _NC * _NS
_K = 2
_BC = _B // _K
_BPW = _BC // _NW
_H = _BPW // 2

_sc_mesh = plsc.VectorSubcoreMesh(core_axis_name="c", subcore_axis_name="s")


def _make_sc_gather(chunk_off):
    @functools.partial(
        pl.kernel,
        mesh=_sc_mesh,
        out_type=jax.ShapeDtypeStruct((_BC, _DIM), jnp.float32),
        scratch_types=[
            pltpu.VMEM((_BPW,), jnp.int32),
            pltpu.VMEM((_BPW, _DIM), jnp.float32),
            pltpu.SemaphoreType.DMA,
            pltpu.SemaphoreType.DMA,
        ],
    )
    def _sc_gather(table_hbm, idx_hbm, out_hbm, idx_v, rows_v, sem_g, sem_w):
        wid = lax.axis_index("s") * _NC + lax.axis_index("c")
        base = wid * _BPW
        pltpu.sync_copy(idx_hbm.at[pl.ds(chunk_off + base, _BPW)], idx_v)
        g0 = pltpu.async_copy(
            table_hbm.at[idx_v.at[pl.ds(0, _H)]], rows_v.at[pl.ds(0, _H)], sem_g)
        g0.wait()
        w0 = pltpu.async_copy(
            rows_v.at[pl.ds(0, _H)], out_hbm.at[pl.ds(base, _H)], sem_w)
        g1 = pltpu.async_copy(
            table_hbm.at[idx_v.at[pl.ds(_H, _H)]], rows_v.at[pl.ds(_H, _H)], sem_g)
        g1.wait()
        w1 = pltpu.async_copy(
            rows_v.at[pl.ds(_H, _H)], out_hbm.at[pl.ds(base + _H, _H)], sem_w)
        w0.wait()
        w1.wait()

    return _sc_gather


_sc_gathers = [_make_sc_gather(k * _BC) for k in range(_K)]

_TC_BLK = 2048


def _tc_body(x_ref, w_ref, b_ref, o_ref):
    acc = lax.dot_general(
        x_ref[...], w_ref[...],
        dimension_numbers=(((1,), (1,)), ((), ())),
        preferred_element_type=jnp.float32,
    )
    o_ref[...] = jnp.tanh(acc + b_ref[...])


def _tc_body_alias(x_ref, w_ref, b_ref, prev_ref, o_ref):
    del prev_ref
    _tc_body(x_ref, w_ref, b_ref, o_ref)


def _tc_chunk(x, w, b2d, prev, k):
    nblk = _BC // _TC_BLK
    off = k * nblk
    in_specs = [
        pl.BlockSpec((_TC_BLK, _DIM), lambda i: (i, 0)),
        pl.BlockSpec((_DIM, _DIM), lambda i: (0, 0)),
        pl.BlockSpec((1, _DIM), lambda i: (0, 0)),
    ]
    args = (x, w, b2d)
    if prev is None:
        body, aliases = _tc_body, {}
    else:
        body, aliases = _tc_body_alias, {3: 0}
        in_specs.append(pl.BlockSpec(memory_space=pl.ANY))
        args = args + (prev,)
    return pl.pallas_call(
        body,
        grid=(nblk,),
        in_specs=in_specs,
        out_specs=pl.BlockSpec((_TC_BLK, _DIM), lambda i, off=off: (i + off, 0)),
        out_shape=jax.ShapeDtypeStruct((_B, _DIM), jnp.float32),
        input_output_aliases=aliases,
    )(*args)


def kernel(nodes, embeddings, W1_w, W1_b):
    idx = nodes.astype(jnp.int32)
    b2d = W1_b.reshape(1, _DIM)
    gathered = [_sc_gathers[k](embeddings, idx) for k in range(_K)]
    out = None
    for k in range(_K):
        out = _tc_chunk(gathered[k], W1_w, b2d, out, k)
    return out


# speedup vs baseline: 1.0444x; 1.0444x over previous
import functools

import jax
import jax.numpy as jnp
from jax import lax
from jax.experimental import pallas as pl
from jax.experimental.pallas import tpu as pltpu
from jax.experimental.pallas import tpu_sc as plsc

_B = 16384
_DIM = 128
_NC = 2
_NS = 16
_NW = _NC * _NS
_BPW = _B // _NW
_H = _BPW // 2

_sc_mesh = plsc.VectorSubcoreMesh(core_axis_name="c", subcore_axis_name="s")


@functools.partial(
    pl.kernel,
    mesh=_sc_mesh,
    out_type=jax.ShapeDtypeStruct((_B, _DIM), jnp.float32),
    scratch_types=[
        pltpu.VMEM((_BPW,), jnp.int32),
        pltpu.VMEM((_BPW, _DIM), jnp.float32),
        pltpu.SemaphoreType.DMA,
        pltpu.SemaphoreType.DMA,
    ],
)
def _sc_gather(table_hbm, idx_hbm, out_hbm, idx_v, rows_v, sem_g, sem_w):
    wid = lax.axis_index("s") * _NC + lax.axis_index("c")
    base = wid * _BPW
    pltpu.sync_copy(idx_hbm.at[pl.ds(base, _BPW)], idx_v)
    g0 = pltpu.async_copy(
        table_hbm.at[idx_v.at[pl.ds(0, _H)]], rows_v.at[pl.ds(0, _H)], sem_g)
    g0.wait()
    w0 = pltpu.async_copy(
        rows_v.at[pl.ds(0, _H)], out_hbm.at[pl.ds(base, _H)], sem_w)
    g1 = pltpu.async_copy(
        table_hbm.at[idx_v.at[pl.ds(_H, _H)]], rows_v.at[pl.ds(_H, _H)], sem_g)
    g1.wait()
    w1 = pltpu.async_copy(
        rows_v.at[pl.ds(_H, _H)], out_hbm.at[pl.ds(base + _H, _H)], sem_w)
    w0.wait()
    w1.wait()


def _tc_body(x_ref, w_ref, b_ref, o_ref):
    acc = lax.dot_general(
        x_ref[...], w_ref[...],
        dimension_numbers=(((1,), (1,)), ((), ())),
        preferred_element_type=jnp.float32,
    )
    o_ref[...] = jnp.tanh(acc + b_ref[...])


def _tc_linear_tanh(x, w, b2d):
    blk = 2048
    return pl.pallas_call(
        _tc_body,
        grid=(_B // blk,),
        in_specs=[
            pl.BlockSpec((blk, _DIM), lambda i: (i, 0)),
            pl.BlockSpec((_DIM, _DIM), lambda i: (0, 0)),
            pl.BlockSpec((1, _DIM), lambda i: (0, 0)),
        ],
        out_specs=pl.BlockSpec((blk, _DIM), lambda i: (i, 0)),
        out_shape=jax.ShapeDtypeStruct((_B, _DIM), jnp.float32),
    )(x, w, b2d)


def kernel(nodes, embeddings, W1_w, W1_b):
    idx = nodes.astype(jnp.int32)
    gathered = _sc_gather(embeddings, idx)
    return _tc_linear_tanh(gathered, W1_w, W1_b.reshape(1, _DIM))


# speedup vs baseline: 1.1545x; 1.1055x over previous
import functools

import jax
import jax.numpy as jnp
from jax import lax
from jax.experimental import pallas as pl
from jax.experimental.pallas import tpu as pltpu
from jax.experimental.pallas import tpu_sc as plsc

_B = 16384
_DIM = 128
_NC = 2
_NS = 16
_NW = _NC * _NS
_BPW = _B // _NW
_H = _BPW // 2

_sc_mesh = plsc.VectorSubcoreMesh(core_axis_name="c", subcore_axis_name="s")


@functools.partial(
    pl.kernel,
    mesh=_sc_mesh,
    out_type=jax.ShapeDtypeStruct((_B, _DIM), jnp.float32),
    scratch_types=[
        pltpu.VMEM((_BPW,), jnp.int32),
        pltpu.VMEM((_BPW, _DIM), jnp.float32),
        pltpu.SemaphoreType.DMA,
        pltpu.SemaphoreType.DMA,
    ],
)
def _sc_gather(table_hbm, idx_hbm, out_hbm, idx_v, rows_v, sem_g, sem_w):
    wid = lax.axis_index("s") * _NC + lax.axis_index("c")
    base = wid * _BPW
    pltpu.sync_copy(idx_hbm.at[pl.ds(base, _BPW)], idx_v)
    pltpu.async_copy(table_hbm.at[idx_v], rows_v, sem_g).wait()
    pltpu.sync_copy(rows_v, out_hbm.at[pl.ds(base, _BPW)])


def _tc_body(x_ref, w_ref, b_ref, o_ref):
    acc = lax.dot_general(
        x_ref[...], w_ref[...],
        dimension_numbers=(((1,), (1,)), ((), ())),
        preferred_element_type=jnp.float32,
    )
    o_ref[...] = jnp.tanh(acc + b_ref[...])


def _tc_linear_tanh(x, w, b2d):
    blk = 4096
    return pl.pallas_call(
        _tc_body,
        grid=(_B // blk,),
        in_specs=[
            pl.BlockSpec((blk, _DIM), lambda i: (i, 0)),
            pl.BlockSpec((_DIM, _DIM), lambda i: (0, 0)),
            pl.BlockSpec((1, _DIM), lambda i: (0, 0)),
        ],
        out_specs=pl.BlockSpec((blk, _DIM), lambda i: (i, 0)),
        out_shape=jax.ShapeDtypeStruct((_B, _DIM), jnp.float32),
    )(x, w, b2d)


def kernel(nodes, embeddings, W1_w, W1_b):
    idx = nodes.astype(jnp.int32)
    gathered = _sc_gather(embeddings, idx)
    return _tc_linear_tanh(gathered, W1_w, W1_b.reshape(1, _DIM))


# speedup vs baseline: 1.2055x; 1.0442x over previous
import functools

import jax
import jax.numpy as jnp
from jax import lax
from jax.experimental import pallas as pl
from jax.experimental.pallas import tpu as pltpu
from jax.experimental.pallas import tpu_sc as plsc

_B = 16384
_DIM = 128
_NC = 2
_NS = 16
_NW = _NC * _NS
_BPW = _B // _NW
_H = _BPW // 2

_sc_mesh = plsc.VectorSubcoreMesh(core_axis_name="c", subcore_axis_name="s")


@functools.partial(
    pl.kernel,
    mesh=_sc_mesh,
    out_type=jax.ShapeDtypeStruct((_B, _DIM), jnp.float32),
    scratch_types=[
        pltpu.VMEM((_BPW,), jnp.int32),
        pltpu.VMEM((_BPW, _DIM), jnp.float32),
        pltpu.SemaphoreType.DMA,
        pltpu.SemaphoreType.DMA,
    ],
)
def _sc_gather(table_hbm, idx_hbm, out_hbm, idx_v, rows_v, sem_g, sem_w):
    wid = lax.axis_index("s") * _NC + lax.axis_index("c")
    base = wid * _BPW
    pltpu.sync_copy(idx_hbm.at[pl.ds(base, _BPW)], idx_v)
    pltpu.async_copy(table_hbm.at[idx_v], rows_v, sem_g).wait()
    pltpu.sync_copy(rows_v, out_hbm.at[pl.ds(base, _BPW)])


def _tc_body(x_ref, w_ref, b_ref, o_ref):
    acc = lax.dot_general(
        x_ref[...], w_ref[...],
        dimension_numbers=(((1,), (1,)), ((), ())),
        preferred_element_type=jnp.float32,
    )
    o_ref[...] = jnp.tanh(acc + b_ref[...])


def _tc_linear_tanh(x, w, b2d):
    blk = 8192
    return pl.pallas_call(
        _tc_body,
        grid=(_B // blk,),
        in_specs=[
            pl.BlockSpec((blk, _DIM), lambda i: (i, 0)),
            pl.BlockSpec((_DIM, _DIM), lambda i: (0, 0)),
            pl.BlockSpec((1, _DIM), lambda i: (0, 0)),
        ],
        out_specs=pl.BlockSpec((blk, _DIM), lambda i: (i, 0)),
        out_shape=jax.ShapeDtypeStruct((_B, _DIM), jnp.float32),
    )(x, w, b2d)


def kernel(nodes, embeddings, W1_w, W1_b):
    idx = nodes.astype(jnp.int32)
    gathered = _sc_gather(embeddings, idx)
    return _tc_linear_tanh(gathered, W1_w, W1_b.reshape(1, _DIM))
